# baseline (speedup 1.0000x reference)
import functools

import jax
import jax.numpy as jnp
from jax import lax
from jax.experimental import pallas as pl
from jax.experimental.pallas import tpu as pltpu
from jax.experimental.pallas import tpu_sc as plsc

N = 10000
E = 320000
D = 128
DC = 8
NC = 2
NS = 16
NW = NC * NS
CH = 80
IG = 8
NGRP = 16
NCHUNK = NGRP * IG
EPT = NCHUNK * CH
E_PAD = EPT * NW
N_ACC = 10112
DUMMY = 10048
ROWS_PER_TILE = N_ACC // NS


def _mlp_body(x_ref, w1_ref, w2_ref, out_ref):
    h = jnp.maximum(jnp.dot(x_ref[...], w1_ref[...],
                            preferred_element_type=jnp.float32), 0.0)
    out_ref[...] = jnp.dot(h, w2_ref[...], preferred_element_type=jnp.float32)


def _node_mlp(x, W1, W2):
    BM = 2000
    return pl.pallas_call(
        _mlp_body,
        grid=(N // BM,),
        in_specs=[pl.BlockSpec((BM, D), lambda i: (i, 0)),
                  pl.BlockSpec((D, D), lambda i: (0, 0)),
                  pl.BlockSpec((D, D), lambda i: (0, 0))],
        out_specs=pl.BlockSpec((BM, D), lambda i: (i, 0)),
        out_shape=jax.ShapeDtypeStruct((N, D), jnp.float32),
    )(x, W1, W2)


def _seg_body(g_hbm, row_hbm, col_hbm, zf_hbm, zc_hbm, ones_hbm,
              feat_out, cnt_out,
              rows0, rows1, ridx0, cidx0, ridx1, cidx1, ones_v,
              acc_f, acc_c, sem_g, sem_s, sem_i):
    cid = lax.axis_index("c")
    sid = lax.axis_index("s")
    wid = sid * NC + cid

    zbase = sid * ROWS_PER_TILE
    pltpu.sync_copy(zf_hbm.at[pl.ds(zbase, ROWS_PER_TILE)],
                    acc_f.at[pl.ds(zbase, ROWS_PER_TILE)])
    pltpu.sync_copy(zc_hbm.at[pl.ds(zbase, ROWS_PER_TILE)],
                    acc_c.at[pl.ds(zbase, ROWS_PER_TILE)])
    pltpu.sync_copy(ones_hbm, ones_v)
    pltpu.sync_copy(row_hbm.at[wid, 0], ridx0)
    pltpu.sync_copy(col_hbm.at[wid, 0], cidx0)
    plsc.subcore_barrier()

    rows = (rows0, rows1)
    ridx = (ridx0, ridx1)
    cidx = (cidx0, cidx1)

    def wait_g():
        pltpu.make_async_copy(g_hbm.at[ridx0.at[0]], rows0, sem_g).wait()

    def wait_s():
        pltpu.make_async_copy(rows0, acc_f.at[cidx0.at[0]], sem_s).wait()
        pltpu.make_async_copy(ones_v, acc_c.at[cidx0.at[0]], sem_s).wait()

    def wait_i():
        pltpu.make_async_copy(row_hbm.at[0, 0], ridx0, sem_i).wait()
        pltpu.make_async_copy(col_hbm.at[0, 0], cidx0, sem_i).wait()

    pltpu.async_copy(row_hbm.at[wid, 1], ridx1, sem_i)
    pltpu.async_copy(col_hbm.at[wid, 1], cidx1, sem_i)
    pltpu.async_copy(g_hbm.at[ridx0.at[0]], rows0, sem_g)

    @pl.loop(0, NGRP // 2)
    def _rounds(r):
        for gg in range(2):
            g = r * 2 + gg
            rx, cx = ridx[gg], cidx[gg]
            nrx = ridx[1 - gg]
            for j in range(IG):
                p = j % 2
                wait_g()
                if j > 0:
                    wait_s()
                pltpu.async_copy(rows[p], acc_f.at[cx.at[j]], sem_s, add=True)
                pltpu.async_copy(ones_v, acc_c.at[cx.at[j]], sem_s, add=True)
                if j < IG - 1:
                    pltpu.async_copy(g_hbm.at[rx.at[j + 1]], rows[1 - p],
                                     sem_g)
            wait_s()

            @pl.when(g < NGRP - 1)
            def _next_group():
                wait_i()
                pltpu.async_copy(g_hbm.at[nrx.at[0]], rows[0], sem_g)

            @pl.when(g < NGRP - 2)
            def _prefetch_idx():
                pltpu.async_copy(row_hbm.at[wid, g + 2], rx, sem_i)
                pltpu.async_copy(col_hbm.at[wid, g + 2], cx, sem_i)

    plsc.subcore_barrier()
    pltpu.sync_copy(acc_f.at[pl.ds(zbase, ROWS_PER_TILE)],
                    feat_out.at[cid, pl.ds(zbase, ROWS_PER_TILE)])
    pltpu.sync_copy(acc_c.at[pl.ds(zbase, ROWS_PER_TILE)],
                    cnt_out.at[cid, pl.ds(zbase, ROWS_PER_TILE)])


@functools.cache
def _make_seg_sum():
    return pl.kernel(
        _seg_body,
        out_type=(jax.ShapeDtypeStruct((NC, N_ACC, D), jnp.float32),
                  jax.ShapeDtypeStruct((NC, N_ACC, DC), jnp.float32)),
        mesh=plsc.VectorSubcoreMesh(core_axis_name="c", subcore_axis_name="s",
                                    num_cores=NC, num_subcores=NS),
        scratch_types=[
            pltpu.VMEM((CH, D), jnp.float32),
            pltpu.VMEM((CH, D), jnp.float32),
            pltpu.VMEM((IG, CH), jnp.int32),
            pltpu.VMEM((IG, CH), jnp.int32),
            pltpu.VMEM((IG, CH), jnp.int32),
            pltpu.VMEM((IG, CH), jnp.int32),
            pltpu.VMEM((CH, DC), jnp.float32),
            pltpu.VMEM_SHARED((N_ACC, D), jnp.float32),
            pltpu.VMEM_SHARED((N_ACC, DC), jnp.float32),
            pltpu.SemaphoreType.DMA,
            pltpu.SemaphoreType.DMA,
            pltpu.SemaphoreType.DMA,
        ],
        compiler_params=pltpu.CompilerParams(use_tc_tiling_on_sc=False),
    )


def _post_body(a0_ref, a1_ref, c0_ref, c1_ref, x_ref, w_ref,
               g1_ref, b1_ref, g2_ref, b2_ref,
               wo1a_ref, wo1b_ref, wo2_ref, out_ref):
    sums = a0_ref[...] + a1_ref[...]
    cnt = (c0_ref[...] + c1_ref[...])[:, :1]
    agg = sums / jnp.maximum(cnt, 1.0)
    m1 = jnp.mean(agg, axis=-1, keepdims=True)
    v1 = jnp.mean((agg - m1) ** 2, axis=-1, keepdims=True)
    agg_n = (agg - m1) * lax.rsqrt(v1 + 1e-5) * g1_ref[...] + b1_ref[...]
    x = x_ref[...]
    y = x + (x - agg_n) * w_ref[...]
    m2 = jnp.mean(y, axis=-1, keepdims=True)
    v2 = jnp.mean((y - m2) ** 2, axis=-1, keepdims=True)
    fx = (y - m2) * lax.rsqrt(v2 + 1e-5) * g2_ref[...] + b2_ref[...]
    h = jnp.maximum(jnp.dot(fx, wo1a_ref[...], preferred_element_type=jnp.float32)
                    + jnp.dot(agg_n, wo1b_ref[...], preferred_element_type=jnp.float32),
                    0.0)
    out_ref[...] = jnp.dot(h, wo2_ref[...], preferred_element_type=jnp.float32)


def _post(a0, a1, c0, c1, x, w, ln1_g, ln1_b, ln2_g, ln2_b, Wo1a, Wo1b, Wo2):
    BM = 2000
    vec = lambda: pl.BlockSpec((1, D), lambda i: (0, 0))
    mat = lambda: pl.BlockSpec((D, D), lambda i: (0, 0))
    return pl.pallas_call(
        _post_body,
        grid=(N // BM,),
        in_specs=[pl.BlockSpec((BM, D), lambda i: (i, 0)),
                  pl.BlockSpec((BM, D), lambda i: (i, 0)),
                  pl.BlockSpec((BM, DC), lambda i: (i, 0)),
                  pl.BlockSpec((BM, DC), lambda i: (i, 0)),
                  pl.BlockSpec((BM, D), lambda i: (i, 0)),
                  vec(), vec(), vec(), vec(), vec(),
                  mat(), mat(), mat()],
        out_specs=pl.BlockSpec((BM, D), lambda i: (i, 0)),
        out_shape=jax.ShapeDtypeStruct((N, D), jnp.float32),
    )(a0, a1, c0, c1, x, w, ln1_g, ln1_b, ln2_g, ln2_b, Wo1a, Wo1b, Wo2)


def kernel(x, edge_index, W1, W2, w, ln1_g, ln1_b, ln2_g, ln2_b, Wo1, Wo2):
    row = edge_index[0].astype(jnp.int32)
    col = edge_index[1].astype(jnp.int32)
    pad = E_PAD - E
    row_p = jnp.concatenate([row, jnp.zeros((pad,), jnp.int32)])
    col_p = jnp.concatenate([col, jnp.full((pad,), DUMMY, jnp.int32)])
    row_p = row_p.reshape(NW, NGRP, IG, CH)
    col_p = col_p.reshape(NW, NGRP, IG, CH)
    zf = jnp.zeros((N_ACC, D), jnp.float32)
    zc = jnp.zeros((N_ACC, DC), jnp.float32)
    ones8 = jnp.zeros((CH, DC), jnp.float32).at[:, 0].set(1.0)

    g = _node_mlp(x, W1, W2)
    pf, pc = _make_seg_sum()(g, row_p, col_p, zf, zc, ones8)

    return _post(pf[0, :N], pf[1, :N], pc[0, :N], pc[1, :N], x,
                 w.reshape(1, D),
                 ln1_g.reshape(1, D), ln1_b.reshape(1, D),
                 ln2_g.reshape(1, D), ln2_b.reshape(1, D),
                 Wo1[:D], Wo1[D:], Wo2)


# speedup vs baseline: 1.0868x; 1.0868x over previous
import functools

import jax
import jax.numpy as jnp
from jax import lax
from jax.experimental import pallas as pl
from jax.experimental.pallas import tpu as pltpu
from jax.experimental.pallas import tpu_sc as plsc

N = 10000
E = 320000
D = 128
DH = 64
DP = 80
NC = 2
NS = 16
CH = 128
IG = 8
NGRP = 20
NCHUNK = NGRP * IG
EPT = NCHUNK * CH
E_PAD = EPT * NS
K = 4
LEAD = 2
N_ACC = 10112
DUMMY = 10048
ROWS_PER_TILE = N_ACC // NS


def _mlp_body(x_ref, w1_ref, w2_ref, out0_ref, out1_ref):
    h = jnp.maximum(jnp.dot(x_ref[...], w1_ref[...],
                            preferred_element_type=jnp.float32), 0.0)
    h = jnp.dot(h, w2_ref[...], preferred_element_type=jnp.float32)
    b = h.shape[0]
    ones = jnp.ones((b, 1), jnp.float32)
    zeros = jnp.zeros((b, DP - DH - 1), jnp.float32)
    out0_ref[...] = jnp.concatenate([h[:, :DH], ones, zeros], axis=1)
    out1_ref[...] = jnp.concatenate([h[:, DH:], ones, zeros], axis=1)


def _node_mlp(x, W1, W2):
    BM = 2000
    return pl.pallas_call(
        _mlp_body,
        grid=(N // BM,),
        in_specs=[pl.BlockSpec((BM, D), lambda i: (i, 0)),
                  pl.BlockSpec((D, D), lambda i: (0, 0)),
                  pl.BlockSpec((D, D), lambda i: (0, 0))],
        out_specs=[pl.BlockSpec((BM, DP), lambda i: (i, 0)),
                   pl.BlockSpec((BM, DP), lambda i: (i, 0))],
        out_shape=[jax.ShapeDtypeStruct((N, DP), jnp.float32),
                   jax.ShapeDtypeStruct((N, DP), jnp.float32)],
    )(x, W1, W2)


def _seg_body(g_hbm, row_hbm, col_hbm, zero_hbm, feat_out,
              rows0, rows1, rows2, rows3, ridx0, cidx0, ridx1, cidx1,
              acc_f, sg0, sg1, sg2, sg3, ss0, ss1, ss2, ss3, sem_i):
    cid = lax.axis_index("c")
    sid = lax.axis_index("s")

    zbase = sid * ROWS_PER_TILE
    pltpu.sync_copy(zero_hbm.at[pl.ds(zbase, ROWS_PER_TILE)],
                    acc_f.at[pl.ds(zbase, ROWS_PER_TILE)])
    pltpu.sync_copy(row_hbm.at[cid, sid, 0], ridx0)
    pltpu.sync_copy(col_hbm.at[sid, 0], cidx0)
    plsc.subcore_barrier()

    rows = (rows0, rows1, rows2, rows3)
    sem_g = (sg0, sg1, sg2, sg3)
    sem_s = (ss0, ss1, ss2, ss3)
    ridx = (ridx0, ridx1)
    cidx = (cidx0, cidx1)

    def issue_gather(idx_row, b):
        pltpu.async_copy(g_hbm.at[idx_row], rows[b], sem_g[b])

    def issue_scat(idx_row, b):
        pltpu.async_copy(rows[b], acc_f.at[idx_row], sem_s[b], add=True)

    def wait_g(b):
        pltpu.make_async_copy(g_hbm.at[ridx0.at[0]], rows[b], sem_g[b]).wait()

    def wait_s(b):
        pltpu.make_async_copy(rows[b], acc_f.at[cidx0.at[0]], sem_s[b]).wait()

    def wait_i():
        pltpu.make_async_copy(row_hbm.at[0, 0, 0], ridx0, sem_i).wait()
        pltpu.make_async_copy(col_hbm.at[0, 0], cidx0, sem_i).wait()

    issue_gather(ridx0.at[0], 0)
    issue_gather(ridx0.at[1], 1)

    @pl.loop(0, NCHUNK // (2 * IG))
    def _super(r):
        for h in range(2):
            g = r * 2 + h
            for j in range(IG):
                c = g * IG + j
                b = j % K
                wait_g(b)
                issue_scat(cidx[h].at[j], b)
                b2 = (j + K - LEAD) % K
                if h == 0 and j < LEAD:
                    @pl.when(r > 0)
                    def _ws():
                        wait_s(b2)
                else:
                    wait_s(b2)
                if j == LEAD:
                    @pl.when(g < NGRP - 1)
                    def _load_next_idx():
                        pltpu.async_copy(row_hbm.at[cid, sid, g + 1],
                                         ridx[1 - h], sem_i)
                        pltpu.async_copy(col_hbm.at[sid, g + 1],
                                         cidx[1 - h], sem_i)
                if j == IG - LEAD - 1:
                    @pl.when(g < NGRP - 1)
                    def _wi():
                        wait_i()
                jn = j + LEAD
                if jn < IG:
                    issue_gather(ridx[h].at[jn], b2)
                else:

                    @pl.when(g < NGRP - 1)
                    def _gnext():
                        issue_gather(ridx[1 - h].at[jn - IG], b2)

    wait_s((NCHUNK - 2) % K)
    wait_s((NCHUNK - 1) % K)

    plsc.subcore_barrier()
    pltpu.sync_copy(acc_f.at[pl.ds(zbase, ROWS_PER_TILE)],
                    feat_out.at[cid, pl.ds(zbase, ROWS_PER_TILE)])


@functools.cache
def _make_seg_sum():
    return pl.kernel(
        _seg_body,
        out_type=jax.ShapeDtypeStruct((NC, N_ACC, DP), jnp.float32),
        mesh=plsc.VectorSubcoreMesh(core_axis_name="c", subcore_axis_name="s",
                                    num_cores=NC, num_subcores=NS),
        scratch_types=[
            pltpu.VMEM((CH, DP), jnp.float32),
            pltpu.VMEM((CH, DP), jnp.float32),
            pltpu.VMEM((CH, DP), jnp.float32),
            pltpu.VMEM((CH, DP), jnp.float32),
            pltpu.VMEM((IG, CH), jnp.int32),
            pltpu.VMEM((IG, CH), jnp.int32),
            pltpu.VMEM((IG, CH), jnp.int32),
            pltpu.VMEM((IG, CH), jnp.int32),
            pltpu.VMEM_SHARED((N_ACC, DP), jnp.float32),
            pltpu.SemaphoreType.DMA,
            pltpu.SemaphoreType.DMA,
            pltpu.SemaphoreType.DMA,
            pltpu.SemaphoreType.DMA,
            pltpu.SemaphoreType.DMA,
            pltpu.SemaphoreType.DMA,
            pltpu.SemaphoreType.DMA,
            pltpu.SemaphoreType.DMA,
            pltpu.SemaphoreType.DMA,
        ],
        compiler_params=pltpu.CompilerParams(use_tc_tiling_on_sc=False),
    )


def _post_body(p0_ref, p1_ref, x_ref, w_ref,
               g1_ref, b1_ref, g2_ref, b2_ref,
               wo1a_ref, wo1b_ref, wo2_ref, out_ref):
    p0 = p0_ref[...]
    p1 = p1_ref[...]
    sums = jnp.concatenate([p0[:, :DH], p1[:, :DH]], axis=1)
    cnt = p0[:, DH:DH + 1]
    agg = sums / jnp.maximum(cnt, 1.0)
    m1 = jnp.mean(agg, axis=-1, keepdims=True)
    v1 = jnp.mean((agg - m1) ** 2, axis=-1, keepdims=True)
    agg_n = (agg - m1) * lax.rsqrt(v1 + 1e-5) * g1_ref[...] + b1_ref[...]
    x = x_ref[...]
    y = x + (x - agg_n) * w_ref[...]
    m2 = jnp.mean(y, axis=-1, keepdims=True)
    v2 = jnp.mean((y - m2) ** 2, axis=-1, keepdims=True)
    fx = (y - m2) * lax.rsqrt(v2 + 1e-5) * g2_ref[...] + b2_ref[...]
    h = jnp.maximum(jnp.dot(fx, wo1a_ref[...], preferred_element_type=jnp.float32)
                    + jnp.dot(agg_n, wo1b_ref[...], preferred_element_type=jnp.float32),
                    0.0)
    out_ref[...] = jnp.dot(h, wo2_ref[...], preferred_element_type=jnp.float32)


def _post(p0, p1, x, w, ln1_g, ln1_b, ln2_g, ln2_b, Wo1a, Wo1b, Wo2):
    BM = 2000
    vec = lambda: pl.BlockSpec((1, D), lambda i: (0, 0))
    mat = lambda: pl.BlockSpec((D, D), lambda i: (0, 0))
    return pl.pallas_call(
        _post_body,
        grid=(N // BM,),
        in_specs=[pl.BlockSpec((BM, DP), lambda i: (i, 0)),
                  pl.BlockSpec((BM, DP), lambda i: (i, 0)),
                  pl.BlockSpec((BM, D), lambda i: (i, 0)),
                  vec(), vec(), vec(), vec(), vec(),
                  mat(), mat(), mat()],
        out_specs=pl.BlockSpec((BM, D), lambda i: (i, 0)),
        out_shape=jax.ShapeDtypeStruct((N, D), jnp.float32),
    )(p0, p1, x, w, ln1_g, ln1_b, ln2_g, ln2_b, Wo1a, Wo1b, Wo2)


def kernel(x, edge_index, W1, W2, w, ln1_g, ln1_b, ln2_g, ln2_b, Wo1, Wo2):
    row = edge_index[0].astype(jnp.int32)
    col = edge_index[1].astype(jnp.int32)
    pad = E_PAD - E
    row_p = jnp.concatenate([row, jnp.zeros((pad,), jnp.int32)])
    col_p = jnp.concatenate([col, jnp.full((pad,), DUMMY, jnp.int32)])
    row_p = row_p.reshape(NS, NGRP, IG, CH)
    row_h = jnp.stack([row_p, row_p + N])
    col_h = col_p.reshape(NS, NGRP, IG, CH)
    zero = jnp.zeros((N_ACC, DP), jnp.float32)

    g0, g1 = _node_mlp(x, W1, W2)
    gt = jnp.concatenate([g0, g1], axis=0)
    pf = _make_seg_sum()(gt, row_h, col_h, zero)

    return _post(pf[0, :N], pf[1, :N], x,
                 w.reshape(1, D),
                 ln1_g.reshape(1, D), ln1_b.reshape(1, D),
                 ln2_g.reshape(1, D), ln2_b.reshape(1, D),
                 Wo1[:D], Wo1[D:], Wo2)
